# baseline (speedup 1.0000x reference)
import jax
import jax.numpy as jnp
from jax.experimental import pallas as pl
from jax.experimental.pallas import tpu as pltpu

N = 10000
NFEAT = 128
NHID = 16
NCLASS = 8
TR = 320
NP = 10240
NR = NP // TR


def _pass1_kernel(x_ref, adj_ref, W1_ref, b1_ref, W2_ref,
                  emb_ref, s2_ref, q_ref, scale_ref, s1_ref):
    i = pl.program_id(0)

    @pl.when(i == 0)
    def _():
        s1_ref[...] = jnp.dot(x_ref[...], W1_ref[...],
                              preferred_element_type=jnp.float32)

    a = adj_ref[...]
    h = jnp.dot(a, s1_ref[...],
                preferred_element_type=jnp.float32) + b1_ref[...]
    h = jnp.maximum(h, 0.0)
    emb_ref[...] = h
    s2_ref[...] = jnp.dot(h, W2_ref[...], preferred_element_type=jnp.float32)

    amax = jnp.max(a, axis=1, keepdims=True)
    scale_ref[...] = jnp.maximum(amax, 1e-30) * (1.0 / 127.0)
    q_ref[...] = jnp.round(a * (127.0 / jnp.maximum(amax, 1e-30))
                           ).astype(jnp.int8)


def _pass2_kernel(s2_ref, q_ref, scale_ref, b2_ref, out_ref,
                  q2_ref, cs_ref):
    i = pl.program_id(0)

    @pl.when(i == 0)
    def _():
        s2 = s2_ref[...]
        cmax = jnp.max(jnp.abs(s2), axis=0, keepdims=True)
        cs = jnp.maximum(cmax, 1e-30) * (1.0 / 127.0)
        cs_ref[...] = cs
        q2_ref[...] = jnp.round(s2 * (127.0 / jnp.maximum(cmax, 1e-30))
                                ).astype(jnp.int8)

    acc = jnp.dot(q_ref[...], q2_ref[...],
                  preferred_element_type=jnp.int32)
    o = acc.astype(jnp.float32) * scale_ref[...] * cs_ref[...] + b2_ref[...]
    m = jnp.max(o, axis=1, keepdims=True)
    lse = m + jnp.log(jnp.sum(jnp.exp(o - m), axis=1, keepdims=True))
    out_ref[...] = o - lse


@jax.jit
def kernel(x, adj, W1, b1, W2, b2):
    b1r = b1.reshape(1, NHID)
    b2r = b2.reshape(1, NCLASS)
    emb_p, s2_p, q, scales = pl.pallas_call(
        _pass1_kernel,
        grid=(NR,),
        in_specs=[
            pl.BlockSpec((N, NFEAT), lambda i: (0, 0)),
            pl.BlockSpec((TR, N), lambda i: (i, 0)),
            pl.BlockSpec((NFEAT, NHID), lambda i: (0, 0)),
            pl.BlockSpec((1, NHID), lambda i: (0, 0)),
            pl.BlockSpec((NHID, NCLASS), lambda i: (0, 0)),
        ],
        out_specs=[
            pl.BlockSpec((TR, NHID), lambda i: (i, 0)),
            pl.BlockSpec((TR, NCLASS), lambda i: (i, 0)),
            pl.BlockSpec((TR, N), lambda i: (i, 0)),
            pl.BlockSpec((TR, 1), lambda i: (i, 0)),
        ],
        out_shape=[
            jax.ShapeDtypeStruct((NP, NHID), jnp.float32),
            jax.ShapeDtypeStruct((NP, NCLASS), jnp.float32),
            jax.ShapeDtypeStruct((NP, N), jnp.int8),
            jax.ShapeDtypeStruct((NP, 1), jnp.float32),
        ],
        scratch_shapes=[pltpu.VMEM((N, NHID), jnp.float32)],
    )(x, adj, W1, b1r, W2)
    out_p = pl.pallas_call(
        _pass2_kernel,
        grid=(NR,),
        in_specs=[
            pl.BlockSpec((N, NCLASS), lambda i: (0, 0)),
            pl.BlockSpec((TR, N), lambda i: (i, 0)),
            pl.BlockSpec((TR, 1), lambda i: (i, 0)),
            pl.BlockSpec((1, NCLASS), lambda i: (0, 0)),
        ],
        out_specs=pl.BlockSpec((TR, NCLASS), lambda i: (i, 0)),
        out_shape=jax.ShapeDtypeStruct((NP, NCLASS), jnp.float32),
        scratch_shapes=[
            pltpu.VMEM((N, NCLASS), jnp.int8),
            pltpu.VMEM((1, NCLASS), jnp.float32),
        ],
    )(s2_p[:N], q, scales, b2r)
    return out_p[:N], emb_p[:N]


# speedup vs baseline: 1.0608x; 1.0608x over previous
import jax
import jax.numpy as jnp
from jax.experimental import pallas as pl
from jax.experimental.pallas import tpu as pltpu

N = 10000
NFEAT = 128
NHID = 16
NCLASS = 8
TR = 320
NP = 10240
NR = NP // TR


def _pass1_kernel(x_ref, adj_ref, W1_ref, b1_ref, W2_ref,
                  emb_ref, s2_ref, q_ref, scale_ref, s1_ref):
    i = pl.program_id(0)

    @pl.when(i == 0)
    def _():
        s1_ref[...] = jnp.dot(x_ref[...], W1_ref[...],
                              preferred_element_type=jnp.float32)

    a = adj_ref[...]
    h = jnp.dot(a, s1_ref[...],
                preferred_element_type=jnp.float32) + b1_ref[...]
    h = jnp.maximum(h, 0.0)
    emb_ref[...] = h
    s2_ref[...] = jnp.dot(h, W2_ref[...], preferred_element_type=jnp.float32)

    amax = jnp.max(a, axis=1, keepdims=True)
    amax = jnp.maximum(amax, 1e-30)
    scale_ref[...] = amax * (1.0 / 256.0)
    q_ref[...] = (a * (256.0 / amax)).astype(jnp.float8_e4m3fn)


def _pass2_kernel(s2_ref, q_ref, scale_ref, b2_ref, out_ref,
                  q2_ref, cs_ref):
    i = pl.program_id(0)

    @pl.when(i == 0)
    def _():
        s2 = s2_ref[...]
        cmax = jnp.maximum(jnp.max(jnp.abs(s2), axis=0, keepdims=True), 1e-30)
        cs_ref[...] = cmax * (1.0 / 256.0)
        q2_ref[...] = (s2 * (256.0 / cmax)).astype(jnp.float8_e4m3fn)

    acc = jnp.dot(q_ref[...], q2_ref[...],
                  preferred_element_type=jnp.float32)
    o = acc * scale_ref[...] * cs_ref[...] + b2_ref[...]
    m = jnp.max(o, axis=1, keepdims=True)
    lse = m + jnp.log(jnp.sum(jnp.exp(o - m), axis=1, keepdims=True))
    out_ref[...] = o - lse


@jax.jit
def kernel(x, adj, W1, b1, W2, b2):
    b1r = b1.reshape(1, NHID)
    b2r = b2.reshape(1, NCLASS)
    emb_p, s2_p, q, scales = pl.pallas_call(
        _pass1_kernel,
        grid=(NR,),
        in_specs=[
            pl.BlockSpec((N, NFEAT), lambda i: (0, 0)),
            pl.BlockSpec((TR, N), lambda i: (i, 0)),
            pl.BlockSpec((NFEAT, NHID), lambda i: (0, 0)),
            pl.BlockSpec((1, NHID), lambda i: (0, 0)),
            pl.BlockSpec((NHID, NCLASS), lambda i: (0, 0)),
        ],
        out_specs=[
            pl.BlockSpec((TR, NHID), lambda i: (i, 0)),
            pl.BlockSpec((TR, NCLASS), lambda i: (i, 0)),
            pl.BlockSpec((TR, N), lambda i: (i, 0)),
            pl.BlockSpec((TR, 1), lambda i: (i, 0)),
        ],
        out_shape=[
            jax.ShapeDtypeStruct((NP, NHID), jnp.float32),
            jax.ShapeDtypeStruct((NP, NCLASS), jnp.float32),
            jax.ShapeDtypeStruct((NP, N), jnp.float8_e4m3fn),
            jax.ShapeDtypeStruct((NP, 1), jnp.float32),
        ],
        scratch_shapes=[pltpu.VMEM((N, NHID), jnp.float32)],
    )(x, adj, W1, b1r, W2)
    out_p = pl.pallas_call(
        _pass2_kernel,
        grid=(NR,),
        in_specs=[
            pl.BlockSpec((N, NCLASS), lambda i: (0, 0)),
            pl.BlockSpec((TR, N), lambda i: (i, 0)),
            pl.BlockSpec((TR, 1), lambda i: (i, 0)),
            pl.BlockSpec((1, NCLASS), lambda i: (0, 0)),
        ],
        out_specs=pl.BlockSpec((TR, NCLASS), lambda i: (i, 0)),
        out_shape=jax.ShapeDtypeStruct((NP, NCLASS), jnp.float32),
        scratch_shapes=[
            pltpu.VMEM((N, NCLASS), jnp.float8_e4m3fn),
            pltpu.VMEM((1, NCLASS), jnp.float32),
        ],
    )(s2_p[:N], q, scales, b2r)
    return out_p[:N], emb_p[:N]


# speedup vs baseline: 1.1351x; 1.0701x over previous
import jax
import jax.numpy as jnp
from jax.experimental import pallas as pl
from jax.experimental.pallas import tpu as pltpu

N = 10000
NFEAT = 128
NHID = 16
NCLASS = 8
TR = 320
NP = 10240
NR = NP // TR


def _pass1_kernel(x_ref, adj_ref, W1_ref, b1_ref, W2_ref,
                  emb_ref, s2_ref, q_ref, s1_ref):
    i = pl.program_id(0)

    @pl.when(i == 0)
    def _():
        s1_ref[...] = jnp.dot(x_ref[...], W1_ref[...],
                              preferred_element_type=jnp.float32)

    a = adj_ref[...]
    h = jnp.dot(a, s1_ref[...],
                preferred_element_type=jnp.float32) + b1_ref[...]
    h = jnp.maximum(h, 0.0)
    emb_ref[...] = h
    s2_ref[...] = jnp.dot(h, W2_ref[...], preferred_element_type=jnp.float32)
    q_ref[...] = (a * 256.0).astype(jnp.float8_e4m3fn)


def _pass2_kernel(s2_ref, q_ref, b2_ref, out_ref, q2_ref, cs_ref):
    i = pl.program_id(0)

    @pl.when(i == 0)
    def _():
        s2 = s2_ref[...]
        cmax = jnp.maximum(jnp.max(jnp.abs(s2), axis=0, keepdims=True), 1e-30)
        cs_ref[...] = cmax * (1.0 / (256.0 * 256.0))
        q2_ref[...] = (s2 * (256.0 / cmax)).astype(jnp.float8_e4m3fn)

    acc = jnp.dot(q_ref[...], q2_ref[...],
                  preferred_element_type=jnp.float32)
    o = acc * cs_ref[...] + b2_ref[...]
    m = jnp.max(o, axis=1, keepdims=True)
    lse = m + jnp.log(jnp.sum(jnp.exp(o - m), axis=1, keepdims=True))
    out_ref[...] = o - lse


@jax.jit
def kernel(x, adj, W1, b1, W2, b2):
    b1r = b1.reshape(1, NHID)
    b2r = b2.reshape(1, NCLASS)
    emb_p, s2_p, q = pl.pallas_call(
        _pass1_kernel,
        grid=(NR,),
        in_specs=[
            pl.BlockSpec((N, NFEAT), lambda i: (0, 0)),
            pl.BlockSpec((TR, N), lambda i: (i, 0)),
            pl.BlockSpec((NFEAT, NHID), lambda i: (0, 0)),
            pl.BlockSpec((1, NHID), lambda i: (0, 0)),
            pl.BlockSpec((NHID, NCLASS), lambda i: (0, 0)),
        ],
        out_specs=[
            pl.BlockSpec((TR, NHID), lambda i: (i, 0)),
            pl.BlockSpec((TR, NCLASS), lambda i: (i, 0)),
            pl.BlockSpec((TR, N), lambda i: (i, 0)),
        ],
        out_shape=[
            jax.ShapeDtypeStruct((NP, NHID), jnp.float32),
            jax.ShapeDtypeStruct((NP, NCLASS), jnp.float32),
            jax.ShapeDtypeStruct((NP, N), jnp.float8_e4m3fn),
        ],
        scratch_shapes=[pltpu.VMEM((N, NHID), jnp.float32)],
    )(x, adj, W1, b1r, W2)
    out_p = pl.pallas_call(
        _pass2_kernel,
        grid=(NR,),
        in_specs=[
            pl.BlockSpec((N, NCLASS), lambda i: (0, 0)),
            pl.BlockSpec((TR, N), lambda i: (i, 0)),
            pl.BlockSpec((1, NCLASS), lambda i: (0, 0)),
        ],
        out_specs=pl.BlockSpec((TR, NCLASS), lambda i: (i, 0)),
        out_shape=jax.ShapeDtypeStruct((NP, NCLASS), jnp.float32),
        scratch_shapes=[
            pltpu.VMEM((N, NCLASS), jnp.float8_e4m3fn),
            pltpu.VMEM((1, NCLASS), jnp.float32),
        ],
    )(s2_p[:N], q, b2r)
    return out_p[:N], emb_p[:N]


# speedup vs baseline: 1.1821x; 1.0414x over previous
import jax
import jax.numpy as jnp
from jax.experimental import pallas as pl
from jax.experimental.pallas import tpu as pltpu

N = 10000
NFEAT = 128
NHID = 16
NCLASS = 8
TR = 512
NP = 10240
NR = NP // TR


def _pass1_kernel(x_ref, adj_ref, W1_ref, b1_ref, W2_ref,
                  emb_ref, s2_ref, q_ref, s1_ref):
    i = pl.program_id(0)

    @pl.when(i == 0)
    def _():
        s1_ref[...] = jnp.dot(x_ref[...], W1_ref[...],
                              preferred_element_type=jnp.float32)

    a = adj_ref[...]
    h = jnp.dot(a, s1_ref[...],
                preferred_element_type=jnp.float32) + b1_ref[...]
    h = jnp.maximum(h, 0.0)
    emb_ref[...] = h
    s2_ref[...] = jnp.dot(h, W2_ref[...], preferred_element_type=jnp.float32)
    q_ref[...] = (a * 256.0).astype(jnp.float8_e4m3fn)


def _pass2_kernel(s2_ref, q_ref, b2_ref, out_ref, q2_ref, cs_ref):
    i = pl.program_id(0)

    @pl.when(i == 0)
    def _():
        s2 = s2_ref[...]
        cmax = jnp.maximum(jnp.max(jnp.abs(s2), axis=0, keepdims=True), 1e-30)
        cs_ref[...] = cmax * (1.0 / (256.0 * 256.0))
        q2_ref[...] = (s2 * (256.0 / cmax)).astype(jnp.float8_e4m3fn)

    acc = jnp.dot(q_ref[...], q2_ref[...],
                  preferred_element_type=jnp.float32)
    o = acc * cs_ref[...] + b2_ref[...]
    m = jnp.max(o, axis=1, keepdims=True)
    lse = m + jnp.log(jnp.sum(jnp.exp(o - m), axis=1, keepdims=True))
    out_ref[...] = o - lse


@jax.jit
def kernel(x, adj, W1, b1, W2, b2):
    b1r = b1.reshape(1, NHID)
    b2r = b2.reshape(1, NCLASS)
    emb_p, s2_p, q = pl.pallas_call(
        _pass1_kernel,
        grid=(NR,),
        in_specs=[
            pl.BlockSpec((N, NFEAT), lambda i: (0, 0)),
            pl.BlockSpec((TR, N), lambda i: (i, 0)),
            pl.BlockSpec((NFEAT, NHID), lambda i: (0, 0)),
            pl.BlockSpec((1, NHID), lambda i: (0, 0)),
            pl.BlockSpec((NHID, NCLASS), lambda i: (0, 0)),
        ],
        out_specs=[
            pl.BlockSpec((TR, NHID), lambda i: (i, 0)),
            pl.BlockSpec((TR, NCLASS), lambda i: (i, 0)),
            pl.BlockSpec((TR, N), lambda i: (i, 0)),
        ],
        out_shape=[
            jax.ShapeDtypeStruct((NP, NHID), jnp.float32),
            jax.ShapeDtypeStruct((NP, NCLASS), jnp.float32),
            jax.ShapeDtypeStruct((NP, N), jnp.float8_e4m3fn),
        ],
        scratch_shapes=[pltpu.VMEM((N, NHID), jnp.float32)],
        compiler_params=pltpu.CompilerParams(
            vmem_limit_bytes=63 * 1024 * 1024),
    )(x, adj, W1, b1r, W2)
    out_p = pl.pallas_call(
        _pass2_kernel,
        grid=(NR,),
        in_specs=[
            pl.BlockSpec((N, NCLASS), lambda i: (0, 0)),
            pl.BlockSpec((TR, N), lambda i: (i, 0)),
            pl.BlockSpec((1, NCLASS), lambda i: (0, 0)),
        ],
        out_specs=pl.BlockSpec((TR, NCLASS), lambda i: (i, 0)),
        out_shape=jax.ShapeDtypeStruct((NP, NCLASS), jnp.float32),
        scratch_shapes=[
            pltpu.VMEM((N, NCLASS), jnp.float8_e4m3fn),
            pltpu.VMEM((1, NCLASS), jnp.float32),
        ],
    )(s2_p[:N], q, b2r)
    return out_p[:N], emb_p[:N]
